# baseline (speedup 1.0000x reference)
import math as _math

import jax
import jax.numpy as jnp
from jax.experimental import pallas as pl
from jax.experimental.pallas import tpu as pltpu

_B = 256
_D = 768
_G = 1000
_P = 128
_K = 5
_TEMP = 1.6
_MAX_REF = 1000.0
_H_THRESH = _math.sin(_MAX_REF / (2.0 * 6371.0)) ** 2


_W = 32
_CHUNK = (_B * _K) // _W


def _dist_kernel(sc_ref, ob_ref, qb_ref, g2_ref, emb_ref, *rest):
    p_refs = rest[:_W]
    coords_ref = rest[_W]
    out_ref = rest[_W + 1]
    i = pl.program_id(0)
    ii = jax.lax.broadcasted_iota(jnp.int32, (_P, 1), 0)
    lane = jax.lax.broadcasted_iota(jnp.int32, (1, 128), 1)

    def heavy(j):
        idx = j * _CHUNK + i
        r = ob_ref[idx]
        b = qb_ref[idx]
        g2 = g2_ref[idx]
        e = emb_ref[pl.ds(b, 1), :]
        pr = p_refs[j][0]
        diff = pr - e
        sums = jnp.sum(diff * diff, axis=1, keepdims=True)
        crows = coords_ref[pl.ds(g2, 2), :]
        return r, sums, crows

    def tail(state):
        r, sums, crows = state
        minv = jnp.min(sums)
        amin = jnp.min(jnp.where(sums == minv, ii, _P))
        lmask = lane == amin
        ll = jnp.sum(jnp.where(lmask, crows, 0.0), axis=1, keepdims=True)
        lng = ll[0:1, :]
        lat = ll[1:2, :]
        row = jnp.where(lane == 0, minv,
                        jnp.where(lane == 1, lng,
                                  jnp.where(lane == 2, lat, 0.0)))
        out_ref[pl.ds(r, 1), :] = row

    pending = []
    for j in range(_W):
        pending.append(heavy(j))
        if len(pending) > 2:
            tail(pending.pop(0))
    for state in pending:
        tail(state)


def _epilogue_kernel(minsq_ref, lng_ref, lat_ref, cprobs_ref, ip_ref,
                     llh_ref, pid_ref, fprobs_ref):
    minsq = minsq_ref[...]
    lngs = lng_ref[...]
    lats = lat_ref[...]
    cprobs = cprobs_ref[...]
    ip = ip_ref[...]

    td = -jnp.sqrt(minsq + 1e-12)
    z = td / _TEMP
    zmax = jnp.max(z, axis=1, keepdims=True)
    ez = jnp.exp(z - zmax)
    probs = ez / jnp.sum(ez, axis=1, keepdims=True)
    fp = cprobs * probs

    jj = jax.lax.broadcasted_iota(jnp.int32, (_B, _K), 1)

    fmax = jnp.max(fp, axis=1, keepdims=True)
    rg = jnp.min(jnp.where(fp == fmax, jj, _K), axis=1, keepdims=True)
    sel = jj == rg
    r_lng = jnp.sum(jnp.where(sel, lngs, 0.0), axis=1, keepdims=True)
    r_lat = jnp.sum(jnp.where(sel, lats, 0.0), axis=1, keepdims=True)

    r = jnp.pi / 180.0
    lng1 = ip[:, 0:1] * r
    lat1 = ip[:, 1:2] * r
    lng2 = r_lng * r
    lat2 = r_lat * r
    h = (jnp.sin((lat2 - lat1) * 0.5) ** 2
         + jnp.cos(lat1) * jnp.cos(lat2) * jnp.sin((lng2 - lng1) * 0.5) ** 2)
    far = jnp.clip(h, 0.0, 1.0) > _H_THRESH

    fp2 = jnp.where(far, cprobs, fp)
    fmax2 = jnp.max(fp2, axis=1, keepdims=True)
    pid = jnp.min(jnp.where(fp2 == fmax2, jj, _K), axis=1, keepdims=True)
    sel2 = jj == pid
    f_lng = jnp.sum(jnp.where(sel2, lngs, 0.0), axis=1, keepdims=True)
    f_lat = jnp.sum(jnp.where(sel2, lats, 0.0), axis=1, keepdims=True)

    llh_ref[:, 0:1] = f_lng
    llh_ref[:, 1:2] = f_lat
    pid_ref[...] = pid
    fprobs_ref[...] = fp2


def kernel(embedding, initial_preds, candidate_cells, candidate_probs,
           protos, proto_coords):
    if embedding.ndim == 3:
        embedding = embedding.mean(axis=1)
    B, K = _B, _K
    n = B * K
    cand = candidate_cells[:, :K].reshape(-1).astype(jnp.int32)
    order = jnp.arange(n, dtype=jnp.int32)
    sc = jnp.take(cand, order)
    qb = order // _K
    g2 = 2 * sc
    coords_t = proto_coords.transpose(0, 2, 1).reshape(2 * _G, _P)

    grid_spec = pltpu.PrefetchScalarGridSpec(
        num_scalar_prefetch=4,
        grid=(_CHUNK,),
        in_specs=(
            [pl.BlockSpec((B, _D), lambda i, s, o, q, g: (0, 0))]
            + [pl.BlockSpec((1, _P, _D),
                            lambda i, s, o, q, g, j=j:
                            (s[j * _CHUNK + i], 0, 0))
               for j in range(_W)]
            + [pl.BlockSpec((2 * _G, _P), lambda i, s, o, q, g: (0, 0))]
        ),
        out_specs=pl.BlockSpec((n, 128), lambda i, s, o, q, g: (0, 0)),
    )
    out = pl.pallas_call(
        _dist_kernel,
        grid_spec=grid_spec,
        out_shape=jax.ShapeDtypeStruct((n, 128), jnp.float32),
    )(sc, order, qb, g2, embedding, *([protos] * _W), coords_t)

    minsq_bk = out[:, 0].reshape(B, K)
    lngs_bk = out[:, 1].reshape(B, K)
    lats_bk = out[:, 2].reshape(B, K)

    llh, pid, fprobs = pl.pallas_call(
        _epilogue_kernel,
        out_shape=[
            jax.ShapeDtypeStruct((B, 2), jnp.float32),
            jax.ShapeDtypeStruct((B, 1), jnp.int32),
            jax.ShapeDtypeStruct((B, K), jnp.float32),
        ],
    )(minsq_bk, lngs_bk, lats_bk, candidate_probs[:, :K].astype(jnp.float32),
      initial_preds)

    return llh, pid[:, 0], fprobs


# speedup vs baseline: 1.0377x; 1.0377x over previous
import math as _math

import jax
import jax.numpy as jnp
from jax.experimental import pallas as pl
from jax.experimental.pallas import tpu as pltpu

_B = 256
_D = 768
_G = 1000
_P = 128
_K = 5
_TEMP = 1.6
_MAX_REF = 1000.0
_H_THRESH = _math.sin(_MAX_REF / (2.0 * 6371.0)) ** 2


_W = 16
_CHUNK = (_B * _K) // _W


def _dist_kernel(sc_ref, ob_ref, qb_ref, g2_ref, emb_ref, *rest):
    p_refs = rest[:_W]
    coords_ref = rest[_W]
    out_ref = rest[_W + 1]
    i = pl.program_id(0)
    ii = jax.lax.broadcasted_iota(jnp.int32, (_P, 1), 0)
    lane = jax.lax.broadcasted_iota(jnp.int32, (1, 128), 1)

    def heavy(j):
        idx = j * _CHUNK + i
        r = ob_ref[idx]
        b = qb_ref[idx]
        g2 = g2_ref[idx]
        e = emb_ref[pl.ds(b, 1), :]
        pr = p_refs[j][0]
        diff = pr - e
        sums = jnp.sum(diff * diff, axis=1, keepdims=True)
        crows = coords_ref[pl.ds(g2, 2), :]
        return r, sums, crows

    def tail(state):
        r, sums, crows = state
        minv = jnp.min(sums)
        amin = jnp.min(jnp.where(sums == minv, ii, _P))
        lmask = lane == amin
        ll = jnp.sum(jnp.where(lmask, crows, 0.0), axis=1, keepdims=True)
        lng = ll[0:1, :]
        lat = ll[1:2, :]
        row = jnp.where(lane == 0, minv,
                        jnp.where(lane == 1, lng,
                                  jnp.where(lane == 2, lat, 0.0)))
        out_ref[pl.ds(r, 1), :] = row

    pending = []
    for j in range(_W):
        pending.append(heavy(j))
        if len(pending) > 2:
            tail(pending.pop(0))
    for state in pending:
        tail(state)


def _epilogue_kernel(minsq_ref, lng_ref, lat_ref, cprobs_ref, ip_ref,
                     llh_ref, pid_ref, fprobs_ref):
    minsq = minsq_ref[...]
    lngs = lng_ref[...]
    lats = lat_ref[...]
    cprobs = cprobs_ref[...]
    ip = ip_ref[...]

    td = -jnp.sqrt(minsq + 1e-12)
    z = td / _TEMP
    zmax = jnp.max(z, axis=1, keepdims=True)
    ez = jnp.exp(z - zmax)
    probs = ez / jnp.sum(ez, axis=1, keepdims=True)
    fp = cprobs * probs

    jj = jax.lax.broadcasted_iota(jnp.int32, (_B, _K), 1)

    fmax = jnp.max(fp, axis=1, keepdims=True)
    rg = jnp.min(jnp.where(fp == fmax, jj, _K), axis=1, keepdims=True)
    sel = jj == rg
    r_lng = jnp.sum(jnp.where(sel, lngs, 0.0), axis=1, keepdims=True)
    r_lat = jnp.sum(jnp.where(sel, lats, 0.0), axis=1, keepdims=True)

    r = jnp.pi / 180.0
    lng1 = ip[:, 0:1] * r
    lat1 = ip[:, 1:2] * r
    lng2 = r_lng * r
    lat2 = r_lat * r
    h = (jnp.sin((lat2 - lat1) * 0.5) ** 2
         + jnp.cos(lat1) * jnp.cos(lat2) * jnp.sin((lng2 - lng1) * 0.5) ** 2)
    far = jnp.clip(h, 0.0, 1.0) > _H_THRESH

    fp2 = jnp.where(far, cprobs, fp)
    fmax2 = jnp.max(fp2, axis=1, keepdims=True)
    pid = jnp.min(jnp.where(fp2 == fmax2, jj, _K), axis=1, keepdims=True)
    sel2 = jj == pid
    f_lng = jnp.sum(jnp.where(sel2, lngs, 0.0), axis=1, keepdims=True)
    f_lat = jnp.sum(jnp.where(sel2, lats, 0.0), axis=1, keepdims=True)

    llh_ref[:, 0:1] = f_lng
    llh_ref[:, 1:2] = f_lat
    pid_ref[...] = pid
    fprobs_ref[...] = fp2


def kernel(embedding, initial_preds, candidate_cells, candidate_probs,
           protos, proto_coords):
    if embedding.ndim == 3:
        embedding = embedding.mean(axis=1)
    B, K = _B, _K
    n = B * K
    cand = candidate_cells[:, :K].reshape(-1).astype(jnp.int32)
    order = jnp.argsort(cand).astype(jnp.int32)
    sc = jnp.take(cand, order)
    qb = order // _K
    g2 = 2 * sc
    coords_t = proto_coords.transpose(0, 2, 1).reshape(2 * _G, _P)

    grid_spec = pltpu.PrefetchScalarGridSpec(
        num_scalar_prefetch=4,
        grid=(_CHUNK,),
        in_specs=(
            [pl.BlockSpec((B, _D), lambda i, s, o, q, g: (0, 0))]
            + [pl.BlockSpec((1, _P, _D),
                            lambda i, s, o, q, g, j=j:
                            (s[j * _CHUNK + i], 0, 0))
               for j in range(_W)]
            + [pl.BlockSpec((2 * _G, _P), lambda i, s, o, q, g: (0, 0))]
        ),
        out_specs=pl.BlockSpec((n, 128), lambda i, s, o, q, g: (0, 0)),
    )
    out = pl.pallas_call(
        _dist_kernel,
        grid_spec=grid_spec,
        out_shape=jax.ShapeDtypeStruct((n, 128), jnp.float32),
    )(sc, order, qb, g2, embedding, *([protos] * _W), coords_t)

    minsq_bk = out[:, 0].reshape(B, K)
    lngs_bk = out[:, 1].reshape(B, K)
    lats_bk = out[:, 2].reshape(B, K)

    llh, pid, fprobs = pl.pallas_call(
        _epilogue_kernel,
        out_shape=[
            jax.ShapeDtypeStruct((B, 2), jnp.float32),
            jax.ShapeDtypeStruct((B, 1), jnp.int32),
            jax.ShapeDtypeStruct((B, K), jnp.float32),
        ],
    )(minsq_bk, lngs_bk, lats_bk, candidate_probs[:, :K].astype(jnp.float32),
      initial_preds)

    return llh, pid[:, 0], fprobs


# speedup vs baseline: 1.1332x; 1.0920x over previous
import math as _math

import jax
import jax.numpy as jnp
from jax.experimental import pallas as pl
from jax.experimental.pallas import tpu as pltpu

_B = 256
_D = 768
_G = 1000
_P = 128
_K = 5
_TEMP = 1.6
_MAX_REF = 1000.0
_H_THRESH = _math.sin(_MAX_REF / (2.0 * 6371.0)) ** 2


_W = 40
_CHUNK = (_B * _K) // _W


def _dist_kernel(sc_ref, ob_ref, qb_ref, g2_ref, emb_ref, *rest):
    p_refs = rest[:_W]
    coords_ref = rest[_W]
    out_ref = rest[_W + 1]
    i = pl.program_id(0)
    ii = jax.lax.broadcasted_iota(jnp.int32, (_P, 1), 0)
    lane = jax.lax.broadcasted_iota(jnp.int32, (1, 128), 1)

    def heavy(j):
        idx = j * _CHUNK + i
        r = ob_ref[idx]
        b = qb_ref[idx]
        g2 = g2_ref[idx]
        e = emb_ref[pl.ds(b, 1), :]
        pr = p_refs[j][0]
        diff = pr - e
        sums = jnp.sum(diff * diff, axis=1, keepdims=True)
        crows = coords_ref[pl.ds(g2, 2), :]
        return r, sums, crows

    def tail(state):
        r, sums, crows = state
        minv = jnp.min(sums)
        amin = jnp.min(jnp.where(sums == minv, ii, _P))
        lmask = lane == amin
        ll = jnp.sum(jnp.where(lmask, crows, 0.0), axis=1, keepdims=True)
        lng = ll[0:1, :]
        lat = ll[1:2, :]
        row = jnp.where(lane == 0, minv,
                        jnp.where(lane == 1, lng,
                                  jnp.where(lane == 2, lat, 0.0)))
        out_ref[pl.ds(r, 1), :] = row

    pending = []
    for j in range(_W):
        pending.append(heavy(j))
        if len(pending) > 2:
            tail(pending.pop(0))
    for state in pending:
        tail(state)


def _epilogue_kernel(minsq_ref, lng_ref, lat_ref, cprobs_ref, ip_ref,
                     llh_ref, pid_ref, fprobs_ref):
    minsq = minsq_ref[...]
    lngs = lng_ref[...]
    lats = lat_ref[...]
    cprobs = cprobs_ref[...]
    ip = ip_ref[...]

    td = -jnp.sqrt(minsq + 1e-12)
    z = td / _TEMP
    zmax = jnp.max(z, axis=1, keepdims=True)
    ez = jnp.exp(z - zmax)
    probs = ez / jnp.sum(ez, axis=1, keepdims=True)
    fp = cprobs * probs

    jj = jax.lax.broadcasted_iota(jnp.int32, (_B, _K), 1)

    fmax = jnp.max(fp, axis=1, keepdims=True)
    rg = jnp.min(jnp.where(fp == fmax, jj, _K), axis=1, keepdims=True)
    sel = jj == rg
    r_lng = jnp.sum(jnp.where(sel, lngs, 0.0), axis=1, keepdims=True)
    r_lat = jnp.sum(jnp.where(sel, lats, 0.0), axis=1, keepdims=True)

    r = jnp.pi / 180.0
    lng1 = ip[:, 0:1] * r
    lat1 = ip[:, 1:2] * r
    lng2 = r_lng * r
    lat2 = r_lat * r
    h = (jnp.sin((lat2 - lat1) * 0.5) ** 2
         + jnp.cos(lat1) * jnp.cos(lat2) * jnp.sin((lng2 - lng1) * 0.5) ** 2)
    far = jnp.clip(h, 0.0, 1.0) > _H_THRESH

    fp2 = jnp.where(far, cprobs, fp)
    fmax2 = jnp.max(fp2, axis=1, keepdims=True)
    pid = jnp.min(jnp.where(fp2 == fmax2, jj, _K), axis=1, keepdims=True)
    sel2 = jj == pid
    f_lng = jnp.sum(jnp.where(sel2, lngs, 0.0), axis=1, keepdims=True)
    f_lat = jnp.sum(jnp.where(sel2, lats, 0.0), axis=1, keepdims=True)

    llh_ref[:, 0:1] = f_lng
    llh_ref[:, 1:2] = f_lat
    pid_ref[...] = pid
    fprobs_ref[...] = fp2


def kernel(embedding, initial_preds, candidate_cells, candidate_probs,
           protos, proto_coords):
    if embedding.ndim == 3:
        embedding = embedding.mean(axis=1)
    B, K = _B, _K
    n = B * K
    cand = candidate_cells[:, :K].reshape(-1).astype(jnp.int32)
    order = jnp.argsort(cand).astype(jnp.int32)
    sc = jnp.take(cand, order)
    qb = order // _K
    g2 = 2 * sc
    coords_t = proto_coords.transpose(0, 2, 1).reshape(2 * _G, _P)

    grid_spec = pltpu.PrefetchScalarGridSpec(
        num_scalar_prefetch=4,
        grid=(_CHUNK,),
        in_specs=(
            [pl.BlockSpec((B, _D), lambda i, s, o, q, g: (0, 0))]
            + [pl.BlockSpec((1, _P, _D),
                            lambda i, s, o, q, g, j=j:
                            (s[j * _CHUNK + i], 0, 0))
               for j in range(_W)]
            + [pl.BlockSpec((2 * _G, _P), lambda i, s, o, q, g: (0, 0))]
        ),
        out_specs=pl.BlockSpec((n, 128), lambda i, s, o, q, g: (0, 0)),
    )
    out = pl.pallas_call(
        _dist_kernel,
        grid_spec=grid_spec,
        out_shape=jax.ShapeDtypeStruct((n, 128), jnp.float32),
    )(sc, order, qb, g2, embedding, *([protos] * _W), coords_t)

    minsq_bk = out[:, 0].reshape(B, K)
    lngs_bk = out[:, 1].reshape(B, K)
    lats_bk = out[:, 2].reshape(B, K)

    llh, pid, fprobs = pl.pallas_call(
        _epilogue_kernel,
        out_shape=[
            jax.ShapeDtypeStruct((B, 2), jnp.float32),
            jax.ShapeDtypeStruct((B, 1), jnp.int32),
            jax.ShapeDtypeStruct((B, K), jnp.float32),
        ],
    )(minsq_bk, lngs_bk, lats_bk, candidate_probs[:, :K].astype(jnp.float32),
      initial_preds)

    return llh, pid[:, 0], fprobs
